# baseline (speedup 1.0000x reference)
import jax
import jax.numpy as jnp
from jax.experimental import pallas as pl
from jax.experimental.pallas import tpu as pltpu

_CH = 128
_NBUF = 8


def _body(sp_ref, rm_hbm, data_hbm, out_hbm,
          inbuf, outbuf, mask_i32, mask_f32,
          mask_sem, in_sem, out_sem):
    batch = data_hbm.shape[0]
    nsteps = batch // _CH
    sp = sp_ref[0]

    mask_cp = pltpu.make_async_copy(
        rm_hbm.at[pl.ds(sp, 1), :], mask_i32, mask_sem)
    mask_cp.start()

    def load(i, b):
        return pltpu.make_async_copy(
            data_hbm.at[pl.ds(i * _CH, _CH), :], inbuf.at[b], in_sem.at[b])

    def store(i, b):
        return pltpu.make_async_copy(
            outbuf.at[b], out_hbm.at[pl.ds(i * _CH, _CH), :], out_sem.at[b])

    for b in range(min(_NBUF, nsteps)):
        load(b, b).start()

    mask_cp.wait()
    mask_f32[...] = mask_i32[...].astype(jnp.float32)

    for i in range(nsteps):
        b = i % _NBUF
        load(i, b).wait()
        if i >= _NBUF:
            store(i - _NBUF, b).wait()
        outbuf[b] = inbuf[b] * mask_f32[...]
        store(i, b).start()
        if i + _NBUF < nsteps:
            load(i + _NBUF, b).start()

    for i in range(max(0, nsteps - _NBUF), nsteps):
        store(i, i % _NBUF).wait()


def kernel(data, selected_param, region_map):
    batch, size = data.shape
    sp = jnp.asarray(selected_param, jnp.int32).reshape((1,))
    return pl.pallas_call(
        _body,
        in_specs=[
            pl.BlockSpec(memory_space=pltpu.MemorySpace.SMEM),
            pl.BlockSpec(memory_space=pl.ANY),
            pl.BlockSpec(memory_space=pl.ANY),
        ],
        out_specs=pl.BlockSpec(memory_space=pl.ANY),
        out_shape=jax.ShapeDtypeStruct((batch, size), jnp.float32),
        scratch_shapes=[
            pltpu.VMEM((_NBUF, _CH, size), jnp.float32),
            pltpu.VMEM((_NBUF, _CH, size), jnp.float32),
            pltpu.VMEM((1, size), jnp.int32),
            pltpu.VMEM((1, size), jnp.float32),
            pltpu.SemaphoreType.DMA,
            pltpu.SemaphoreType.DMA((_NBUF,)),
            pltpu.SemaphoreType.DMA((_NBUF,)),
        ],
    )(sp, region_map, data)


# speedup vs baseline: 1.0069x; 1.0069x over previous
import jax
import jax.numpy as jnp
from jax.experimental import pallas as pl
from jax.experimental.pallas import tpu as pltpu

_CH = 256
_NBUF = 6


def _body(sp_ref, rm_hbm, data_hbm, out_hbm,
          inbuf, outbuf, mask_i32, mask_f32,
          mask_sem, in_sem, out_sem):
    batch = data_hbm.shape[0]
    nsteps = batch // _CH
    sp = sp_ref[0]

    mask_cp = pltpu.make_async_copy(
        rm_hbm.at[pl.ds(sp, 1), :], mask_i32, mask_sem)
    mask_cp.start()

    def load(i, b):
        return pltpu.make_async_copy(
            data_hbm.at[pl.ds(i * _CH, _CH), :], inbuf.at[b], in_sem.at[b])

    def store(i, b):
        return pltpu.make_async_copy(
            outbuf.at[b], out_hbm.at[pl.ds(i * _CH, _CH), :], out_sem.at[b])

    for b in range(min(_NBUF, nsteps)):
        load(b, b).start()

    mask_cp.wait()
    mask_f32[...] = mask_i32[...].astype(jnp.float32)

    for i in range(nsteps):
        b = i % _NBUF
        load(i, b).wait()
        if i >= _NBUF:
            store(i - _NBUF, b).wait()
        outbuf[b] = inbuf[b] * mask_f32[...]
        store(i, b).start()
        if i + _NBUF < nsteps:
            load(i + _NBUF, b).start()

    for i in range(max(0, nsteps - _NBUF), nsteps):
        store(i, i % _NBUF).wait()


def kernel(data, selected_param, region_map):
    batch, size = data.shape
    sp = jnp.asarray(selected_param, jnp.int32).reshape((1,))
    return pl.pallas_call(
        _body,
        in_specs=[
            pl.BlockSpec(memory_space=pltpu.MemorySpace.SMEM),
            pl.BlockSpec(memory_space=pl.ANY),
            pl.BlockSpec(memory_space=pl.ANY),
        ],
        out_specs=pl.BlockSpec(memory_space=pl.ANY),
        out_shape=jax.ShapeDtypeStruct((batch, size), jnp.float32),
        scratch_shapes=[
            pltpu.VMEM((_NBUF, _CH, size), jnp.float32),
            pltpu.VMEM((_NBUF, _CH, size), jnp.float32),
            pltpu.VMEM((1, size), jnp.int32),
            pltpu.VMEM((1, size), jnp.float32),
            pltpu.SemaphoreType.DMA,
            pltpu.SemaphoreType.DMA((_NBUF,)),
            pltpu.SemaphoreType.DMA((_NBUF,)),
        ],
    )(sp, region_map, data)


# speedup vs baseline: 1.0079x; 1.0010x over previous
import jax
import jax.numpy as jnp
from jax.experimental import pallas as pl
from jax.experimental.pallas import tpu as pltpu

_CH = 256
_NBUF = 7


def _body(sp_ref, rm_hbm, data_hbm, out_hbm,
          inbuf, outbuf, mask_i32, mask_f32,
          mask_sem, in_sem, out_sem):
    batch = data_hbm.shape[0]
    nsteps = batch // _CH
    sp = sp_ref[0]

    mask_cp = pltpu.make_async_copy(
        rm_hbm.at[pl.ds(sp, 1), :], mask_i32, mask_sem)
    mask_cp.start()

    def load(i, b):
        return pltpu.make_async_copy(
            data_hbm.at[pl.ds(i * _CH, _CH), :], inbuf.at[b], in_sem.at[b])

    def store(i, b):
        return pltpu.make_async_copy(
            outbuf.at[b], out_hbm.at[pl.ds(i * _CH, _CH), :], out_sem.at[b])

    for b in range(min(_NBUF, nsteps)):
        load(b, b).start()

    mask_cp.wait()
    mask_f32[...] = mask_i32[...].astype(jnp.float32)

    for i in range(nsteps):
        b = i % _NBUF
        load(i, b).wait()
        if i >= _NBUF:
            store(i - _NBUF, b).wait()
        outbuf[b] = inbuf[b] * mask_f32[...]
        store(i, b).start()
        if i + _NBUF < nsteps:
            load(i + _NBUF, b).start()

    for i in range(max(0, nsteps - _NBUF), nsteps):
        store(i, i % _NBUF).wait()


def kernel(data, selected_param, region_map):
    batch, size = data.shape
    sp = jnp.asarray(selected_param, jnp.int32).reshape((1,))
    return pl.pallas_call(
        _body,
        in_specs=[
            pl.BlockSpec(memory_space=pltpu.MemorySpace.SMEM),
            pl.BlockSpec(memory_space=pl.ANY),
            pl.BlockSpec(memory_space=pl.ANY),
        ],
        out_specs=pl.BlockSpec(memory_space=pl.ANY),
        out_shape=jax.ShapeDtypeStruct((batch, size), jnp.float32),
        scratch_shapes=[
            pltpu.VMEM((_NBUF, _CH, size), jnp.float32),
            pltpu.VMEM((_NBUF, _CH, size), jnp.float32),
            pltpu.VMEM((1, size), jnp.int32),
            pltpu.VMEM((1, size), jnp.float32),
            pltpu.SemaphoreType.DMA,
            pltpu.SemaphoreType.DMA((_NBUF,)),
            pltpu.SemaphoreType.DMA((_NBUF,)),
        ],
    )(sp, region_map, data)
